# baseline (speedup 1.0000x reference)
import functools

import jax
import jax.numpy as jnp
from jax import lax
from jax.experimental import pallas as pl
from jax.experimental.pallas import tpu as pltpu
from jax.experimental.pallas import tpu_sc as plsc

N_NODES = 10000
N_EDGES = 160000
D_IN = 256
D_OUT = 512

H = 128
PADW = 144
NC = 2
NS = 16
B = 128
NB = 79
EPT = NB * B
E_PAD = NS * EPT
AROWS = N_NODES + 8
K = 2
NZT = 10
SLAB = N_NODES // NZT


def _sc_aggregate(xlo, xhi, row3, col3, zrows):
    mesh = plsc.VectorSubcoreMesh(
        core_axis_name="c", subcore_axis_name="s", num_cores=NC,
        num_subcores=NS)

    @functools.partial(
        pl.kernel,
        out_type=jax.ShapeDtypeStruct((NC * N_NODES, PADW), jnp.float32),
        mesh=mesh,
        scratch_types=[
            pltpu.VMEM_SHARED((AROWS, PADW), jnp.float32),
            pltpu.VMEM((K, B), jnp.int32),
            pltpu.VMEM((K, B), jnp.int32),
            pltpu.VMEM((K, B, PADW), jnp.float32),
            pltpu.SemaphoreType.DMA((K,)),
            pltpu.SemaphoreType.DMA((K,)),
            pltpu.SemaphoreType.DMA((K,)),
        ],
        compiler_params=pltpu.CompilerParams(use_tc_tiling_on_sc=False),
    )
    def agg(xlo_hbm, xhi_hbm, row_hbm, col_hbm, z_hbm, out_hbm,
            acc, row_r, col_r, feat_r, isem, gsem, ssem):
        c = lax.axis_index("c")
        s = lax.axis_index("s")

        def start_idx(b, p):
            pltpu.async_copy(row_hbm.at[s].at[b], row_r.at[p], isem.at[p])
            pltpu.async_copy(col_hbm.at[s].at[b], col_r.at[p], isem.at[p])

        def wait_idx(p):
            pltpu.make_async_copy(row_hbm.at[s].at[0], row_r.at[p],
                                  isem.at[p]).wait()
            pltpu.make_async_copy(col_hbm.at[s].at[0], col_r.at[p],
                                  isem.at[p]).wait()

        def start_gather(p):
            @pl.when(c == 0)
            def _():
                pltpu.async_copy(xlo_hbm.at[col_r.at[p]], feat_r.at[p],
                                 gsem.at[p])

            @pl.when(c != 0)
            def _():
                pltpu.async_copy(xhi_hbm.at[col_r.at[p]], feat_r.at[p],
                                 gsem.at[p])

        def wait_gather(p):
            pltpu.make_async_copy(xlo_hbm.at[col_r.at[p]], feat_r.at[p],
                                  gsem.at[p]).wait()

        def wait_scatter(p):
            pltpu.make_async_copy(feat_r.at[p], acc.at[row_r.at[p]],
                                  ssem.at[p]).wait()

        for p in range(K):
            start_idx(p, p)

        @pl.when(s < NZT)
        def _():
            pltpu.sync_copy(z_hbm, acc.at[pl.ds(s * SLAB, SLAB)])
        plsc.subcore_barrier()
        wait_idx(0)
        start_gather(0)

        def body(b, carry):
            p = lax.rem(b, K)
            q = lax.rem(b + 1, K)

            @pl.when(b + 1 < NB)
            def _():
                wait_idx(q)
                start_gather(q)
            wait_gather(p)
            pltpu.async_copy(feat_r.at[p], acc.at[row_r.at[p]],
                             ssem.at[p], add=True)

            @pl.when(b + K < NB)
            def _():
                wait_scatter(p)
                start_idx(b + K, p)
            return carry

        lax.fori_loop(0, NB, body, 0)
        for p in range(K):
            wait_scatter(p)
        plsc.subcore_barrier()

        @pl.when(s < NZT)
        def _():
            pltpu.sync_copy(acc.at[pl.ds(s * SLAB, SLAB)],
                            out_hbm.at[pl.ds(c * N_NODES + s * SLAB, SLAB)])

    return agg(xlo, xhi, row3, col3, zrows)


def _tc_combine(x, agg, W_self, W_nei, b_self2, b_nei2):
    R = 1000

    def body(x_ref, agg_ref, ws_ref, wn_ref, bs_ref, bn_ref, o_ref):
        a = agg_ref[...]
        cnt = a[0, :, H:H + 1]
        inv = 1.0 / (cnt + 1e-12)
        nlo = a[0, :, :H] * inv
        nhi = a[1, :, :H] * inv
        xb = x_ref[...]
        wn = wn_ref[...]
        dn = (((1,), (1,)), ((), ()))
        h = lax.dot_general(xb, ws_ref[...], dn,
                            preferred_element_type=jnp.float32)
        h = h + lax.dot_general(nlo, wn[:, :H], dn,
                                preferred_element_type=jnp.float32)
        h = h + lax.dot_general(nhi, wn[:, H:], dn,
                                preferred_element_type=jnp.float32)
        h = h + bs_ref[...] + bn_ref[...]
        o_ref[...] = jnp.maximum(h, 0.0)

    return pl.pallas_call(
        body,
        grid=(N_NODES // R,),
        in_specs=[
            pl.BlockSpec((R, D_IN), lambda i: (i, 0)),
            pl.BlockSpec((2, R, PADW), lambda i: (0, i, 0)),
            pl.BlockSpec((D_OUT, D_IN), lambda i: (0, 0)),
            pl.BlockSpec((D_OUT, D_IN), lambda i: (0, 0)),
            pl.BlockSpec((1, D_OUT), lambda i: (0, 0)),
            pl.BlockSpec((1, D_OUT), lambda i: (0, 0)),
        ],
        out_specs=pl.BlockSpec((R, D_OUT), lambda i: (i, 0)),
        out_shape=jax.ShapeDtypeStruct((N_NODES, D_OUT), jnp.float32),
    )(x, agg, W_self, W_nei, b_self2, b_nei2)


def kernel(x, edge_index, W_self, b_self, W_nei, b_nei):
    row = edge_index[0].astype(jnp.int32)
    col = edge_index[1].astype(jnp.int32)
    ones = jnp.ones((N_NODES, 1), jnp.float32)
    pad = jnp.zeros((N_NODES, PADW - H - 1), jnp.float32)
    xlo = jnp.concatenate([x[:, :H], ones, pad], axis=1)
    xhi = jnp.concatenate([x[:, H:], ones, pad], axis=1)
    zrows = jnp.zeros((SLAB, PADW), jnp.float32)
    npad = E_PAD - N_EDGES
    row3 = jnp.concatenate(
        [row, jnp.full((npad,), N_NODES, jnp.int32)]).reshape(NS, NB, B)
    col3 = jnp.concatenate(
        [col, jnp.zeros((npad,), jnp.int32)]).reshape(NS, NB, B)
    agg = _sc_aggregate(xlo, xhi, row3, col3, zrows).reshape(2, N_NODES, PADW)
    return _tc_combine(x, agg, W_self, W_nei,
                       b_self.reshape(1, D_OUT), b_nei.reshape(1, D_OUT))


# speedup vs baseline: 1.0460x; 1.0460x over previous
import functools

import jax
import jax.numpy as jnp
from jax import lax
from jax.experimental import pallas as pl
from jax.experimental.pallas import tpu as pltpu
from jax.experimental.pallas import tpu_sc as plsc

N_NODES = 10000
N_EDGES = 160000
D_IN = 256
D_OUT = 512

H = 128
PADW = 144
NC = 2
NS = 16
B = 128
NB = 79
EPT = NB * B
E_PAD = NS * EPT
AROWS = N_NODES + 8
K = 2
IK = 4
NZT = 10
SLAB = N_NODES // NZT


def _sc_aggregate(xlo, xhi, row3, col3, zrows):
    mesh = plsc.VectorSubcoreMesh(
        core_axis_name="c", subcore_axis_name="s", num_cores=NC,
        num_subcores=NS)

    @functools.partial(
        pl.kernel,
        out_type=jax.ShapeDtypeStruct((NC * N_NODES, PADW), jnp.float32),
        mesh=mesh,
        scratch_types=[
            pltpu.VMEM_SHARED((AROWS, PADW), jnp.float32),
            pltpu.VMEM((IK, B), jnp.int32),
            pltpu.VMEM((IK, B), jnp.int32),
            pltpu.VMEM((K, B, PADW), jnp.float32),
            pltpu.SemaphoreType.DMA((IK,)),
            pltpu.SemaphoreType.DMA((K,)),
            pltpu.SemaphoreType.DMA((K,)),
        ],
        compiler_params=pltpu.CompilerParams(use_tc_tiling_on_sc=False),
    )
    def agg(xlo_hbm, xhi_hbm, row_hbm, col_hbm, z_hbm, out_hbm,
            acc, row_r, col_r, feat_r, isem, gsem, ssem):
        c = lax.axis_index("c")
        s = lax.axis_index("s")

        def start_idx(b, p):
            pltpu.async_copy(row_hbm.at[s].at[b], row_r.at[p], isem.at[p])
            pltpu.async_copy(col_hbm.at[s].at[b], col_r.at[p], isem.at[p])

        def wait_idx(p):
            pltpu.make_async_copy(row_hbm.at[s].at[0], row_r.at[p],
                                  isem.at[p]).wait()
            pltpu.make_async_copy(col_hbm.at[s].at[0], col_r.at[p],
                                  isem.at[p]).wait()

        def start_gather(p, ip):
            @pl.when(c == 0)
            def _():
                pltpu.async_copy(xlo_hbm.at[col_r.at[ip]], feat_r.at[p],
                                 gsem.at[p])

            @pl.when(c != 0)
            def _():
                pltpu.async_copy(xhi_hbm.at[col_r.at[ip]], feat_r.at[p],
                                 gsem.at[p])

        def wait_gather(p):
            pltpu.make_async_copy(xlo_hbm.at[col_r.at[0]], feat_r.at[p],
                                  gsem.at[p]).wait()

        def wait_scatter(p):
            pltpu.make_async_copy(feat_r.at[p], acc.at[row_r.at[0]],
                                  ssem.at[p]).wait()

        for p in range(IK):
            start_idx(p, p)

        @pl.when(s < NZT)
        def _():
            pltpu.sync_copy(z_hbm, acc.at[pl.ds(s * SLAB, SLAB)])
        plsc.subcore_barrier()
        wait_idx(0)
        start_gather(0, 0)

        def body(b, carry):
            p = lax.rem(b, K)
            ip = lax.rem(b, IK)

            @pl.when(b + 1 < NB)
            def _():
                iq = lax.rem(b + 1, IK)
                wait_idx(iq)
                start_gather(lax.rem(b + 1, K), iq)
            wait_gather(p)
            pltpu.async_copy(feat_r.at[p], acc.at[row_r.at[ip]],
                             ssem.at[p], add=True)

            @pl.when(b + K < NB)
            def _():
                wait_scatter(p)

                @pl.when(b + IK < NB)
                def _():
                    start_idx(b + IK, ip)
            return carry

        lax.fori_loop(0, NB, body, 0)
        for p in range(K):
            wait_scatter(p)
        plsc.subcore_barrier()

        @pl.when(s < NZT)
        def _():
            pltpu.sync_copy(acc.at[pl.ds(s * SLAB, SLAB)],
                            out_hbm.at[pl.ds(c * N_NODES + s * SLAB, SLAB)])

    return agg(xlo, xhi, row3, col3, zrows)


def _tc_combine(x, agg, W_self, W_nei, b_self2, b_nei2):
    R = 1000

    def body(x_ref, agg_ref, ws_ref, wn_ref, bs_ref, bn_ref, o_ref):
        a = agg_ref[...]
        cnt = a[0, :, H:H + 1]
        inv = 1.0 / (cnt + 1e-12)
        nlo = a[0, :, :H] * inv
        nhi = a[1, :, :H] * inv
        xb = x_ref[...]
        wn = wn_ref[...]
        dn = (((1,), (1,)), ((), ()))
        h = lax.dot_general(xb, ws_ref[...], dn,
                            preferred_element_type=jnp.float32)
        h = h + lax.dot_general(nlo, wn[:, :H], dn,
                                preferred_element_type=jnp.float32)
        h = h + lax.dot_general(nhi, wn[:, H:], dn,
                                preferred_element_type=jnp.float32)
        h = h + bs_ref[...] + bn_ref[...]
        o_ref[...] = jnp.maximum(h, 0.0)

    return pl.pallas_call(
        body,
        grid=(N_NODES // R,),
        in_specs=[
            pl.BlockSpec((R, D_IN), lambda i: (i, 0)),
            pl.BlockSpec((2, R, PADW), lambda i: (0, i, 0)),
            pl.BlockSpec((D_OUT, D_IN), lambda i: (0, 0)),
            pl.BlockSpec((D_OUT, D_IN), lambda i: (0, 0)),
            pl.BlockSpec((1, D_OUT), lambda i: (0, 0)),
            pl.BlockSpec((1, D_OUT), lambda i: (0, 0)),
        ],
        out_specs=pl.BlockSpec((R, D_OUT), lambda i: (i, 0)),
        out_shape=jax.ShapeDtypeStruct((N_NODES, D_OUT), jnp.float32),
    )(x, agg, W_self, W_nei, b_self2, b_nei2)


def kernel(x, edge_index, W_self, b_self, W_nei, b_nei):
    row = edge_index[0].astype(jnp.int32)
    col = edge_index[1].astype(jnp.int32)
    ones = jnp.ones((N_NODES, 1), jnp.float32)
    pad = jnp.zeros((N_NODES, PADW - H - 1), jnp.float32)
    xlo = jnp.concatenate([x[:, :H], ones, pad], axis=1)
    xhi = jnp.concatenate([x[:, H:], ones, pad], axis=1)
    zrows = jnp.zeros((SLAB, PADW), jnp.float32)
    npad = E_PAD - N_EDGES
    row3 = jnp.concatenate(
        [row, jnp.full((npad,), N_NODES, jnp.int32)]).reshape(NS, NB, B)
    col3 = jnp.concatenate(
        [col, jnp.zeros((npad,), jnp.int32)]).reshape(NS, NB, B)
    agg = _sc_aggregate(xlo, xhi, row3, col3, zrows).reshape(2, N_NODES, PADW)
    return _tc_combine(x, agg, W_self, W_nei,
                       b_self.reshape(1, D_OUT), b_nei.reshape(1, D_OUT))


# speedup vs baseline: 1.5624x; 1.4936x over previous
import functools

import jax
import jax.numpy as jnp
from jax import lax
from jax.experimental import pallas as pl
from jax.experimental.pallas import tpu as pltpu
from jax.experimental.pallas import tpu_sc as plsc

N_NODES = 10000
N_EDGES = 160000
D_IN = 256
D_OUT = 512

H = 128
PADW = 144
NC = 2
NS = 16
B = 80
EPT = N_EDGES // NS
NB = EPT // B
AROWS = N_NODES + 8
K = 3
IK = 6
NZT = 10
SLAB = N_NODES // NZT


def _sc_aggregate(xlo, xhi, row3, col3, zrows):
    mesh = plsc.VectorSubcoreMesh(
        core_axis_name="c", subcore_axis_name="s", num_cores=NC,
        num_subcores=NS)

    @functools.partial(
        pl.kernel,
        out_type=jax.ShapeDtypeStruct((NC * N_NODES, PADW), jnp.float32),
        mesh=mesh,
        scratch_types=[
            pltpu.VMEM_SHARED((AROWS, PADW), jnp.float32),
            pltpu.VMEM((IK, B), jnp.int32),
            pltpu.VMEM((IK, B), jnp.int32),
            pltpu.VMEM((K, B, PADW), jnp.float32),
            pltpu.SemaphoreType.DMA((IK,)),
            pltpu.SemaphoreType.DMA((K,)),
            pltpu.SemaphoreType.DMA((K,)),
        ],
        compiler_params=pltpu.CompilerParams(use_tc_tiling_on_sc=False),
    )
    def agg(xlo_hbm, xhi_hbm, row_hbm, col_hbm, z_hbm, out_hbm,
            acc, row_r, col_r, feat_r, isem, gsem, ssem):
        c = lax.axis_index("c")
        s = lax.axis_index("s")

        def start_idx(b, p):
            pltpu.async_copy(row_hbm.at[s].at[b], row_r.at[p], isem.at[p])
            pltpu.async_copy(col_hbm.at[s].at[b], col_r.at[p], isem.at[p])

        def wait_idx(p):
            pltpu.make_async_copy(row_hbm.at[s].at[0], row_r.at[p],
                                  isem.at[p]).wait()
            pltpu.make_async_copy(col_hbm.at[s].at[0], col_r.at[p],
                                  isem.at[p]).wait()

        def start_gather(p, ip):
            @pl.when(c == 0)
            def _():
                pltpu.async_copy(xlo_hbm.at[col_r.at[ip]], feat_r.at[p],
                                 gsem.at[p])

            @pl.when(c != 0)
            def _():
                pltpu.async_copy(xhi_hbm.at[col_r.at[ip]], feat_r.at[p],
                                 gsem.at[p])

        def wait_gather(p):
            pltpu.make_async_copy(xlo_hbm.at[col_r.at[0]], feat_r.at[p],
                                  gsem.at[p]).wait()

        def wait_scatter(p):
            pltpu.make_async_copy(feat_r.at[p], acc.at[row_r.at[0]],
                                  ssem.at[p]).wait()

        for p in range(IK):
            start_idx(p, p)

        @pl.when(s < NZT)
        def _():
            pltpu.sync_copy(z_hbm, acc.at[pl.ds(s * SLAB, SLAB)])
        plsc.subcore_barrier()
        for t in range(2):
            wait_idx(t)
            start_gather(t, t)

        def body(b, carry):
            p = lax.rem(b, K)
            ip = lax.rem(b, IK)

            @pl.when(b + 2 < NB)
            def _():
                iq = lax.rem(b + 2, IK)
                wait_idx(iq)
                start_gather(lax.rem(b + 2, K), iq)
            wait_gather(p)
            pltpu.async_copy(feat_r.at[p], acc.at[row_r.at[ip]],
                             ssem.at[p], add=True)

            @pl.when(b + K < NB)
            def _():
                wait_scatter(p)

                @pl.when(b + IK < NB)
                def _():
                    start_idx(b + IK, ip)
            return carry

        lax.fori_loop(0, NB, body, 0)
        for p in range(K):
            wait_scatter(p)
        plsc.subcore_barrier()

        @pl.when(s < NZT)
        def _():
            pltpu.sync_copy(acc.at[pl.ds(s * SLAB, SLAB)],
                            out_hbm.at[pl.ds(c * N_NODES + s * SLAB, SLAB)])

    return agg(xlo, xhi, row3, col3, zrows)


def _tc_combine(x, agg, W_self, W_nei, b_self2, b_nei2):
    R = 1000

    def body(x_ref, agg_ref, ws_ref, wn_ref, bs_ref, bn_ref, o_ref):
        a = agg_ref[...]
        cnt = a[0, :, H:H + 1]
        inv = 1.0 / (cnt + 1e-12)
        nlo = a[0, :, :H] * inv
        nhi = a[1, :, :H] * inv
        xb = x_ref[...]
        wn = wn_ref[...]
        dn = (((1,), (1,)), ((), ()))
        h = lax.dot_general(xb, ws_ref[...], dn,
                            preferred_element_type=jnp.float32)
        h = h + lax.dot_general(nlo, wn[:, :H], dn,
                                preferred_element_type=jnp.float32)
        h = h + lax.dot_general(nhi, wn[:, H:], dn,
                                preferred_element_type=jnp.float32)
        h = h + bs_ref[...] + bn_ref[...]
        o_ref[...] = jnp.maximum(h, 0.0)

    return pl.pallas_call(
        body,
        grid=(N_NODES // R,),
        in_specs=[
            pl.BlockSpec((R, D_IN), lambda i: (i, 0)),
            pl.BlockSpec((2, R, PADW), lambda i: (0, i, 0)),
            pl.BlockSpec((D_OUT, D_IN), lambda i: (0, 0)),
            pl.BlockSpec((D_OUT, D_IN), lambda i: (0, 0)),
            pl.BlockSpec((1, D_OUT), lambda i: (0, 0)),
            pl.BlockSpec((1, D_OUT), lambda i: (0, 0)),
        ],
        out_specs=pl.BlockSpec((R, D_OUT), lambda i: (i, 0)),
        out_shape=jax.ShapeDtypeStruct((N_NODES, D_OUT), jnp.float32),
    )(x, agg, W_self, W_nei, b_self2, b_nei2)


def kernel(x, edge_index, W_self, b_self, W_nei, b_nei):
    row = edge_index[0].astype(jnp.int32)
    col = edge_index[1].astype(jnp.int32)
    ones = jnp.ones((N_NODES, 1), jnp.float32)
    pad = jnp.zeros((N_NODES, PADW - H - 1), jnp.float32)
    xlo = jnp.concatenate([x[:, :H], ones, pad], axis=1)
    xhi = jnp.concatenate([x[:, H:], ones, pad], axis=1)
    zrows = jnp.zeros((SLAB, PADW), jnp.float32)
    row3 = row.reshape(NS, NB, B)
    col3 = col.reshape(NS, NB, B)
    agg = _sc_aggregate(xlo, xhi, row3, col3, zrows).reshape(2, N_NODES, PADW)
    return _tc_combine(x, agg, W_self, W_nei,
                       b_self.reshape(1, D_OUT), b_nei.reshape(1, D_OUT))
